# baseline (speedup 1.0000x reference)
import functools

import jax
import jax.numpy as jnp
from jax import lax
from jax.experimental import pallas as pl
from jax.experimental.pallas import tpu as pltpu
from jax.experimental.pallas import tpu_sc as plsc

N = 10000
E = 320000
D = 128
DH = D // 2
NC = 2
NS = 16
NW = NC * NS
EPW = E // NW
CHUNK = 125
NCH = EPW // CHUNK
NBUF = 8
LEAD = 4
NPAD = 10240
RPT = NPAD // NS
ZROWS = 64

_mesh = plsc.VectorSubcoreMesh(core_axis_name="c", subcore_axis_name="s")


def _worker_id():
    return lax.axis_index("c") * NS + lax.axis_index("s")


def _zero_fill(zbuf, rows, cols):
    def body(i, _):
        r = i // (cols // 16)
        c0 = (i % (cols // 16)) * 16
        zbuf[r, pl.ds(c0, 16)] = jnp.zeros((16,), jnp.float32)
        return 0
    lax.fori_loop(0, rows * (cols // 16), body, 0)


@functools.partial(
    pl.kernel,
    out_type=(jax.ShapeDtypeStruct((NPAD,), jnp.float32),
              jax.ShapeDtypeStruct((NPAD,), jnp.float32)),
    mesh=_mesh,
    scratch_types=[
        pltpu.VMEM((NCH, CHUNK), jnp.int32),
        pltpu.VMEM((128,), jnp.float32),
        pltpu.VMEM((RPT,), jnp.float32),
        pltpu.VMEM_SHARED((NPAD,), jnp.float32),
        pltpu.SemaphoreType.DMA,
    ],
)
def _deg_kernel(ei_hbm, deg0_hbm, deg1_hbm, didx, ones, zbuf, acc, sem):
    c = lax.axis_index("c")
    s = lax.axis_index("s")
    w = _worker_id()

    def fill_z(i, _):
        zbuf[pl.ds(i * 16, 16)] = jnp.zeros((16,), jnp.float32)
        return 0
    lax.fori_loop(0, RPT // 16, fill_z, 0)

    def fill_ones(i, _):
        ones[pl.ds(i * 16, 16)] = jnp.full((16,), 1.0, jnp.float32)
        return 0
    lax.fori_loop(0, 8, fill_ones, 0)

    pltpu.sync_copy(ei_hbm.at[1, w], didx)
    pltpu.sync_copy(zbuf, acc.at[pl.ds(s * RPT, RPT)])
    plsc.subcore_barrier()

    def fire(j, _):
        pltpu.async_copy(ones.at[pl.ds(0, CHUNK)], acc.at[didx.at[j]], sem,
                         add=True)
        return 0
    lax.fori_loop(0, NCH, fire, 0)

    def drain(j, _):
        pltpu.make_async_copy(ones.at[pl.ds(0, CHUNK)], acc.at[didx.at[0]],
                              sem).wait()
        return 0
    lax.fori_loop(0, NCH, drain, 0)
    plsc.subcore_barrier()

    @pl.when(c == 0)
    def _():
        pltpu.sync_copy(acc.at[pl.ds(s * RPT, RPT)],
                        deg0_hbm.at[pl.ds(s * RPT, RPT)])

    @pl.when(c == 1)
    def _():
        pltpu.sync_copy(acc.at[pl.ds(s * RPT, RPT)],
                        deg1_hbm.at[pl.ds(s * RPT, RPT)])


@functools.partial(
    pl.kernel,
    out_type=(jax.ShapeDtypeStruct((NPAD, D), jnp.float32),
              jax.ShapeDtypeStruct((NPAD, D), jnp.float32)),
    mesh=_mesh,
    scratch_types=[
        pltpu.VMEM((NCH, CHUNK), jnp.int32),
        pltpu.VMEM((NCH, CHUNK), jnp.int32),
        pltpu.VMEM((NBUF, CHUNK, DH), jnp.float32),
        pltpu.VMEM((ZROWS, DH), jnp.float32),
        pltpu.VMEM_SHARED((NPAD, DH), jnp.float32),
        [pltpu.SemaphoreType.DMA] * NBUF,
        [pltpu.SemaphoreType.DMA] * NBUF,
    ],
    compiler_params=pltpu.CompilerParams(use_tc_tiling_on_sc=False),
)
def _agg_kernel(hsa_hbm, hsb_hbm, ei_hbm, o0_hbm, o1_hbm,
                sidx, didx, bufs, zbuf, acc, gsems, ssems):
    c = lax.axis_index("c")
    s = lax.axis_index("s")
    w = _worker_id()

    _zero_fill(zbuf, ZROWS, DH)
    pltpu.sync_copy(ei_hbm.at[0, w], sidx)
    pltpu.sync_copy(ei_hbm.at[1, w], didx)

    for hs_hbm, col0 in ((hsa_hbm, 0), (hsb_hbm, DH)):
        def zero_acc(k, _):
            pltpu.sync_copy(zbuf, acc.at[pl.ds(s * RPT + k * ZROWS, ZROWS)])
            return 0
        lax.fori_loop(0, RPT // ZROWS, zero_acc, 0)
        plsc.subcore_barrier()

        for b in range(LEAD):
            pltpu.async_copy(hs_hbm.at[sidx.at[b]], bufs.at[b], gsems[b])

        def body(jj, _):
            for b in range(NBUF):
                j = jj * NBUF + b
                bg = (b + LEAD) % NBUF

                @pl.when(j - (NBUF - LEAD) >= 0)
                def _():
                    pltpu.make_async_copy(bufs.at[bg],
                                          acc.at[didx.at[0]],
                                          ssems[bg]).wait()

                @pl.when(j + LEAD < NCH)
                def _():
                    pltpu.async_copy(hs_hbm.at[sidx.at[j + LEAD]],
                                     bufs.at[bg], gsems[bg])

                pltpu.make_async_copy(hs_hbm.at[sidx.at[j]], bufs.at[b],
                                      gsems[b]).wait()
                pltpu.async_copy(bufs.at[b], acc.at[didx.at[j]], ssems[b],
                                 add=True)
            return 0
        lax.fori_loop(0, NCH // NBUF, body, 0)

        for j in range(NCH - (NBUF - LEAD), NCH):
            b = j % NBUF
            pltpu.make_async_copy(bufs.at[b], acc.at[didx.at[0]],
                                  ssems[b]).wait()
        plsc.subcore_barrier()

        @pl.when(c == 0)
        def _():
            pltpu.sync_copy(acc.at[pl.ds(s * RPT, RPT)],
                            o0_hbm.at[pl.ds(s * RPT, RPT), pl.ds(col0, DH)])

        @pl.when(c == 1)
        def _():
            pltpu.sync_copy(acc.at[pl.ds(s * RPT, RPT)],
                            o1_hbm.at[pl.ds(s * RPT, RPT), pl.ds(col0, DH)])


_RB = 1024
_NB = (N + _RB - 1) // _RB


def _dis_col(d0_ref, d1_ref):
    deg = d0_ref[...] + d1_ref[...] + 1.0
    dis = lax.rsqrt(deg)
    r = lax.broadcasted_iota(jnp.int32, (_RB, _RB), 0)
    c = lax.broadcasted_iota(jnp.int32, (_RB, _RB), 1)
    ident = jnp.where(r == c, 1.0, 0.0).astype(jnp.float32)
    return lax.dot_general(ident, dis, (((1,), (1,)), ((), ())),
                           preferred_element_type=jnp.float32)


def _t1_body(x_ref, w_ref, d0_ref, d1_ref, b_ref, hsa_ref, hsb_ref, base_ref):
    h = lax.dot_general(x_ref[...], w_ref[...], (((1,), (1,)), ((), ())),
                        preferred_element_type=jnp.float32)
    dis = _dis_col(d0_ref, d1_ref)
    hs = h * dis
    hsa_ref[...] = hs[:, :DH]
    hsb_ref[...] = hs[:, DH:]
    base_ref[...] = h * (dis * dis) + b_ref[...]


def _t2_body(o0_ref, o1_ref, d0_ref, d1_ref, base_ref, y_ref):
    dis = _dis_col(d0_ref, d1_ref)
    y_ref[...] = (o0_ref[...] + o1_ref[...]) * dis + base_ref[...]


def _t1(x, W, d0, d1, b2):
    return pl.pallas_call(
        _t1_body,
        out_shape=(jax.ShapeDtypeStruct((N, DH), jnp.float32),
                   jax.ShapeDtypeStruct((N, DH), jnp.float32),
                   jax.ShapeDtypeStruct((N, D), jnp.float32)),
        grid=(_NB,),
        in_specs=[
            pl.BlockSpec((_RB, D), lambda i: (i, 0)),
            pl.BlockSpec((D, D), lambda i: (0, 0)),
            pl.BlockSpec((1, _RB), lambda i: (0, i)),
            pl.BlockSpec((1, _RB), lambda i: (0, i)),
            pl.BlockSpec((1, D), lambda i: (0, 0)),
        ],
        out_specs=(pl.BlockSpec((_RB, DH), lambda i: (i, 0)),
                   pl.BlockSpec((_RB, DH), lambda i: (i, 0)),
                   pl.BlockSpec((_RB, D), lambda i: (i, 0))),
    )(x, W, d0, d1, b2)


def _t2(o0, o1, d0, d1, base):
    return pl.pallas_call(
        _t2_body,
        out_shape=jax.ShapeDtypeStruct((N, D), jnp.float32),
        grid=(_NB,),
        in_specs=[
            pl.BlockSpec((_RB, D), lambda i: (i, 0)),
            pl.BlockSpec((_RB, D), lambda i: (i, 0)),
            pl.BlockSpec((1, _RB), lambda i: (0, i)),
            pl.BlockSpec((1, _RB), lambda i: (0, i)),
            pl.BlockSpec((_RB, D), lambda i: (i, 0)),
        ],
        out_specs=pl.BlockSpec((_RB, D), lambda i: (i, 0)),
    )(o0, o1, d0, d1, base)


def kernel(x, edge_index, W, b):
    ei4 = edge_index.astype(jnp.int32).reshape(2, NW, NCH, CHUNK)

    deg0, deg1 = _deg_kernel(ei4)
    d0 = deg0.reshape(1, NPAD)
    d1 = deg1.reshape(1, NPAD)
    b2 = b.reshape(1, D)

    hsa, hsb, base = _t1(x, W, d0, d1, b2)
    o0, o1 = _agg_kernel(hsa, hsb, ei4)
    return _t2(o0, o1, d0, d1, base)


# speedup vs baseline: 1.0182x; 1.0182x over previous
import functools

import jax
import jax.numpy as jnp
from jax import lax
from jax.experimental import pallas as pl
from jax.experimental.pallas import tpu as pltpu
from jax.experimental.pallas import tpu_sc as plsc

N = 10000
E = 320000
D = 128
DH = D // 2
NC = 2
NS = 16
NW = NC * NS
EPW = E // NW
CHUNK = 125
NCH = EPW // CHUNK
NBUF = 5
LEAD = 3
NPAD = 10240
RPT = NPAD // NS
ZROWS = 64

_mesh = plsc.VectorSubcoreMesh(core_axis_name="c", subcore_axis_name="s")


def _worker_id():
    return lax.axis_index("c") * NS + lax.axis_index("s")


def _zero_fill(zbuf, rows, cols):
    def body(i, _):
        r = i // (cols // 16)
        c0 = (i % (cols // 16)) * 16
        zbuf[r, pl.ds(c0, 16)] = jnp.zeros((16,), jnp.float32)
        return 0
    lax.fori_loop(0, rows * (cols // 16), body, 0)


@functools.partial(
    pl.kernel,
    out_type=(jax.ShapeDtypeStruct((NPAD,), jnp.float32),
              jax.ShapeDtypeStruct((NPAD,), jnp.float32)),
    mesh=_mesh,
    scratch_types=[
        pltpu.VMEM((NCH, CHUNK), jnp.int32),
        pltpu.VMEM((128,), jnp.float32),
        pltpu.VMEM((RPT,), jnp.float32),
        pltpu.VMEM_SHARED((NPAD,), jnp.float32),
        pltpu.SemaphoreType.DMA,
    ],
)
def _deg_kernel(ei_hbm, deg0_hbm, deg1_hbm, didx, ones, zbuf, acc, sem):
    c = lax.axis_index("c")
    s = lax.axis_index("s")
    w = _worker_id()

    def fill_z(i, _):
        zbuf[pl.ds(i * 16, 16)] = jnp.zeros((16,), jnp.float32)
        return 0
    lax.fori_loop(0, RPT // 16, fill_z, 0)

    def fill_ones(i, _):
        ones[pl.ds(i * 16, 16)] = jnp.full((16,), 1.0, jnp.float32)
        return 0
    lax.fori_loop(0, 8, fill_ones, 0)

    pltpu.sync_copy(ei_hbm.at[1, w], didx)
    pltpu.sync_copy(zbuf, acc.at[pl.ds(s * RPT, RPT)])
    plsc.subcore_barrier()

    def fire(j, _):
        pltpu.async_copy(ones.at[pl.ds(0, CHUNK)], acc.at[didx.at[j]], sem,
                         add=True)
        return 0
    lax.fori_loop(0, NCH, fire, 0)

    def drain(j, _):
        pltpu.make_async_copy(ones.at[pl.ds(0, CHUNK)], acc.at[didx.at[0]],
                              sem).wait()
        return 0
    lax.fori_loop(0, NCH, drain, 0)
    plsc.subcore_barrier()

    @pl.when(c == 0)
    def _():
        pltpu.sync_copy(acc.at[pl.ds(s * RPT, RPT)],
                        deg0_hbm.at[pl.ds(s * RPT, RPT)])

    @pl.when(c == 1)
    def _():
        pltpu.sync_copy(acc.at[pl.ds(s * RPT, RPT)],
                        deg1_hbm.at[pl.ds(s * RPT, RPT)])


@functools.partial(
    pl.kernel,
    out_type=(jax.ShapeDtypeStruct((NPAD, D), jnp.float32),
              jax.ShapeDtypeStruct((NPAD, D), jnp.float32)),
    mesh=_mesh,
    scratch_types=[
        pltpu.VMEM((NCH, CHUNK), jnp.int32),
        pltpu.VMEM((NCH, CHUNK), jnp.int32),
        pltpu.VMEM((NBUF, CHUNK, DH), jnp.float32),
        pltpu.VMEM((ZROWS, DH), jnp.float32),
        pltpu.VMEM_SHARED((NPAD, DH), jnp.float32),
        [pltpu.SemaphoreType.DMA] * NBUF,
        [pltpu.SemaphoreType.DMA] * NBUF,
    ],
    compiler_params=pltpu.CompilerParams(use_tc_tiling_on_sc=False),
)
def _agg_kernel(hsa_hbm, hsb_hbm, ei_hbm, o0_hbm, o1_hbm,
                sidx, didx, bufs, zbuf, acc, gsems, ssems):
    c = lax.axis_index("c")
    s = lax.axis_index("s")
    w = _worker_id()

    _zero_fill(zbuf, ZROWS, DH)
    pltpu.sync_copy(ei_hbm.at[0, w], sidx)
    pltpu.sync_copy(ei_hbm.at[1, w], didx)

    for hs_hbm, col0 in ((hsa_hbm, 0), (hsb_hbm, DH)):
        def zero_acc(k, _):
            pltpu.sync_copy(zbuf, acc.at[pl.ds(s * RPT + k * ZROWS, ZROWS)])
            return 0
        lax.fori_loop(0, RPT // ZROWS, zero_acc, 0)
        plsc.subcore_barrier()

        for b in range(LEAD):
            pltpu.async_copy(hs_hbm.at[sidx.at[b]], bufs.at[b], gsems[b])

        def body(jj, _):
            for b in range(NBUF):
                j = jj * NBUF + b
                bg = (b + LEAD) % NBUF

                @pl.when(j - (NBUF - LEAD) >= 0)
                def _():
                    pltpu.make_async_copy(bufs.at[bg],
                                          acc.at[didx.at[0]],
                                          ssems[bg]).wait()

                @pl.when(j + LEAD < NCH)
                def _():
                    pltpu.async_copy(hs_hbm.at[sidx.at[j + LEAD]],
                                     bufs.at[bg], gsems[bg])

                pltpu.make_async_copy(hs_hbm.at[sidx.at[j]], bufs.at[b],
                                      gsems[b]).wait()
                pltpu.async_copy(bufs.at[b], acc.at[didx.at[j]], ssems[b],
                                 add=True)
            return 0
        lax.fori_loop(0, NCH // NBUF, body, 0)

        for j in range(NCH - (NBUF - LEAD), NCH):
            b = j % NBUF
            pltpu.make_async_copy(bufs.at[b], acc.at[didx.at[0]],
                                  ssems[b]).wait()
        plsc.subcore_barrier()

        @pl.when(c == 0)
        def _():
            pltpu.sync_copy(acc.at[pl.ds(s * RPT, RPT)],
                            o0_hbm.at[pl.ds(s * RPT, RPT), pl.ds(col0, DH)])

        @pl.when(c == 1)
        def _():
            pltpu.sync_copy(acc.at[pl.ds(s * RPT, RPT)],
                            o1_hbm.at[pl.ds(s * RPT, RPT), pl.ds(col0, DH)])


_RB = 1024
_NB = (N + _RB - 1) // _RB


def _dis_col(d0_ref, d1_ref):
    deg = d0_ref[...] + d1_ref[...] + 1.0
    dis = lax.rsqrt(deg)
    r = lax.broadcasted_iota(jnp.int32, (_RB, _RB), 0)
    c = lax.broadcasted_iota(jnp.int32, (_RB, _RB), 1)
    ident = jnp.where(r == c, 1.0, 0.0).astype(jnp.float32)
    return lax.dot_general(ident, dis, (((1,), (1,)), ((), ())),
                           preferred_element_type=jnp.float32)


def _t1_body(x_ref, w_ref, d0_ref, d1_ref, b_ref, hsa_ref, hsb_ref, base_ref):
    h = lax.dot_general(x_ref[...], w_ref[...], (((1,), (1,)), ((), ())),
                        preferred_element_type=jnp.float32)
    dis = _dis_col(d0_ref, d1_ref)
    hs = h * dis
    hsa_ref[...] = hs[:, :DH]
    hsb_ref[...] = hs[:, DH:]
    base_ref[...] = h * (dis * dis) + b_ref[...]


def _t2_body(o0_ref, o1_ref, d0_ref, d1_ref, base_ref, y_ref):
    dis = _dis_col(d0_ref, d1_ref)
    y_ref[...] = (o0_ref[...] + o1_ref[...]) * dis + base_ref[...]


def _t1(x, W, d0, d1, b2):
    return pl.pallas_call(
        _t1_body,
        out_shape=(jax.ShapeDtypeStruct((N, DH), jnp.float32),
                   jax.ShapeDtypeStruct((N, DH), jnp.float32),
                   jax.ShapeDtypeStruct((N, D), jnp.float32)),
        grid=(_NB,),
        in_specs=[
            pl.BlockSpec((_RB, D), lambda i: (i, 0)),
            pl.BlockSpec((D, D), lambda i: (0, 0)),
            pl.BlockSpec((1, _RB), lambda i: (0, i)),
            pl.BlockSpec((1, _RB), lambda i: (0, i)),
            pl.BlockSpec((1, D), lambda i: (0, 0)),
        ],
        out_specs=(pl.BlockSpec((_RB, DH), lambda i: (i, 0)),
                   pl.BlockSpec((_RB, DH), lambda i: (i, 0)),
                   pl.BlockSpec((_RB, D), lambda i: (i, 0))),
    )(x, W, d0, d1, b2)


def _t2(o0, o1, d0, d1, base):
    return pl.pallas_call(
        _t2_body,
        out_shape=jax.ShapeDtypeStruct((N, D), jnp.float32),
        grid=(_NB,),
        in_specs=[
            pl.BlockSpec((_RB, D), lambda i: (i, 0)),
            pl.BlockSpec((_RB, D), lambda i: (i, 0)),
            pl.BlockSpec((1, _RB), lambda i: (0, i)),
            pl.BlockSpec((1, _RB), lambda i: (0, i)),
            pl.BlockSpec((_RB, D), lambda i: (i, 0)),
        ],
        out_specs=pl.BlockSpec((_RB, D), lambda i: (i, 0)),
    )(o0, o1, d0, d1, base)


def kernel(x, edge_index, W, b):
    ei4 = edge_index.astype(jnp.int32).reshape(2, NW, NCH, CHUNK)

    deg0, deg1 = _deg_kernel(ei4)
    d0 = deg0.reshape(1, NPAD)
    d1 = deg1.reshape(1, NPAD)
    b2 = b.reshape(1, D)

    hsa, hsb, base = _t1(x, W, d0, d1, b2)
    o0, o1 = _agg_kernel(hsa, hsb, ei4)
    return _t2(o0, o1, d0, d1, base)
